# Initial kernel scaffold; baseline (speedup 1.0000x reference)
import jax
import jax.numpy as jnp
from jax.experimental import pallas as pl


def kernel(x):
    raise NotImplementedError("write your pallas kernel here")


# speedup vs baseline: 19.4366x; 19.4366x over previous
import functools

import jax
import jax.numpy as jnp
from jax import lax
from jax.experimental import pallas as pl
from jax.experimental.pallas import tpu as pltpu
from jax.experimental.pallas import tpu_sc as plsc

_N = 16
_CLEAN = 8
_D = 128
_B = 16384
_S = _B // _N
_SBLK = 64
_GRID = _S // _SBLK

_NW = 32
_RPW = (_S * _CLEAN) // _NW
_CH = 128
_NCH = _RPW // _CH
_LANES = 16


def _score_body(x_ref, out_ref):
    xb = x_ref[...]
    xs = xb.reshape(_SBLK, _N, _D)
    sim = lax.dot_general(
        xs, xs, (((2,), (2,)), ((0,), (0,))),
        preferred_element_type=jnp.float32)
    a = sim[:, :, None, :]
    b = sim[:, :, :, None]
    m_io = lax.broadcasted_iota(jnp.int32, (_SBLK, _N, _N, _N), 3)
    k_io = lax.broadcasted_iota(jnp.int32, (_SBLK, _N, _N, _N), 2)
    ind = (a < b) | ((a == b) & (m_io < k_io))
    part = jnp.sum(ind.astype(jnp.int32), axis=(0, 1, 3))

    @pl.when(pl.program_id(0) == 0)
    def _():
        out_ref[...] = jnp.zeros_like(out_ref)

    out_ref[...] += part[None, :]


def _scores(x):
    out = pl.pallas_call(
        _score_body,
        grid=(_GRID,),
        in_specs=[pl.BlockSpec((_SBLK * _N, _D), lambda i: (i, 0))],
        out_specs=pl.BlockSpec((1, _N), lambda i: (0, 0)),
        out_shape=jax.ShapeDtypeStruct((1, _N), jnp.int32),
    )(x)
    return out.reshape(_N)


def _make_select_gather():
    mesh = plsc.VectorSubcoreMesh(core_axis_name="c", subcore_axis_name="s")

    @functools.partial(
        pl.kernel, mesh=mesh,
        compiler_params=pltpu.CompilerParams(needs_layout_passes=False),
        out_type=jax.ShapeDtypeStruct((_S * _CLEAN, _D), jnp.float32),
        scratch_types=[
            pltpu.VMEM((_N,), jnp.int32),
            pltpu.VMEM((_N,), jnp.int32),
            pltpu.VMEM((_N,), jnp.int32),
            pltpu.VMEM((_NCH, _CH), jnp.int32),
            pltpu.VMEM((_RPW, _D), jnp.float32),
            pltpu.SemaphoreType.DMA,
        ],
    )
    def select_gather(x_hbm, scores_hbm, out_hbm, sc_v, key_v, sel_v, idx_v,
                      rows_v, sem):
        wid = lax.axis_index("s") * 2 + lax.axis_index("c")
        base = wid * _RPW
        pltpu.sync_copy(scores_hbm, sc_v)
        lane = lax.iota(jnp.int32, _LANES)
        key = sc_v[...] * _N + (_N - 1 - lane)
        key_v[...] = key
        pos = jnp.zeros((_LANES,), jnp.int32)
        for m in range(_N):
            km = plsc.load_gather(key_v, [jnp.full((_LANES,), m, jnp.int32)])
            pos += (km > key).astype(jnp.int32)
        plsc.store_scatter(sel_v, [pos], lane)
        for i in range(_RPW // _LANES):
            t = base + i * _LANES + lane
            c = jnp.bitwise_and(t, _CLEAN - 1)
            j = lax.shift_right_logical(t, 3)
            src = j * _N + plsc.load_gather(sel_v, [c])
            off = i * _LANES
            idx_v[off // _CH, pl.ds(off % _CH, _LANES)] = src
        copies = [
            pltpu.async_copy(x_hbm.at[idx_v.at[ci]],
                             rows_v.at[pl.ds(ci * _CH, _CH)], sem)
            for ci in range(_NCH)
        ]
        for cp in copies:
            cp.wait()
        pltpu.sync_copy(rows_v, out_hbm.at[pl.ds(base, _RPW)])

    return select_gather


@functools.cache
def _select_gather_fn():
    return _make_select_gather()


def kernel(x):
    scores = _scores(x)
    return _select_gather_fn()(x, scores)


# speedup vs baseline: 42.7480x; 2.1994x over previous
import functools

import jax
import jax.numpy as jnp
import numpy as np
from jax import lax
from jax.experimental import pallas as pl
from jax.experimental.pallas import tpu as pltpu
from jax.experimental.pallas import tpu_sc as plsc

_N = 16
_CLEAN = 8
_D = 128
_B = 16384
_S = _B // _N
_SBLK = 64
_GRID = _S // _SBLK

_NW = 32
_RPW = (_S * _CLEAN) // _NW
_CH = 128
_NCH = _RPW // _CH
_LANES = 16


def _score_body(x_ref, out_ref):
    r_io = lax.broadcasted_iota(jnp.int32, (_N, _N * _N), 0)
    p_io = lax.broadcasted_iota(jnp.int32, (_N, _N * _N), 1)
    ma = (p_io % _N == r_io).astype(jnp.float32)
    mb = (p_io // _N == r_io).astype(jnp.float32)
    q_io = lax.broadcasted_iota(jnp.int32, (1, _N * _N), 1)
    cm = (q_io % _N) < (q_io // _N)

    xb = x_ref[...]
    xs = xb.reshape(_SBLK, _N, _D)
    sim = lax.dot_general(
        xs, xs, (((2,), (2,)), ((0,), (0,))),
        preferred_element_type=jnp.float32)
    simf = sim.reshape(_SBLK * _N, _N)
    a = jnp.dot(simf, ma, preferred_element_type=jnp.float32)
    b = jnp.dot(simf, mb, preferred_element_type=jnp.float32)
    ind = (a < b) | ((a == b) & cm)
    part = jnp.sum(ind.astype(jnp.int32), axis=0)

    @pl.when(pl.program_id(0) == 0)
    def _():
        out_ref[...] = jnp.zeros_like(out_ref)

    out_ref[...] += part[None, :]


def _scores(x):
    out = pl.pallas_call(
        _score_body,
        grid=(_GRID,),
        in_specs=[pl.BlockSpec((_SBLK * _N, _D), lambda i: (i, 0))],
        out_specs=pl.BlockSpec((1, _N * _N), lambda i: (0, 0)),
        out_shape=jax.ShapeDtypeStruct((1, _N * _N), jnp.int32),
    )(x)
    return out.reshape(_N * _N)


def _make_select_gather():
    mesh = plsc.VectorSubcoreMesh(core_axis_name="c", subcore_axis_name="s")

    @functools.partial(
        pl.kernel, mesh=mesh,
        compiler_params=pltpu.CompilerParams(needs_layout_passes=False),
        out_type=jax.ShapeDtypeStruct((_S * _CLEAN, _D), jnp.float32),
        scratch_types=[
            pltpu.VMEM((_N * _N,), jnp.int32),
            pltpu.VMEM((_N,), jnp.int32),
            pltpu.VMEM((_N,), jnp.int32),
            pltpu.VMEM((_NCH, _CH), jnp.int32),
            pltpu.VMEM((_RPW, _D), jnp.float32),
            pltpu.SemaphoreType.DMA,
        ],
    )
    def select_gather(x_hbm, scores_hbm, out_hbm, sc_v, key_v, sel_v, idx_v,
                      rows_v, sem):
        wid = lax.axis_index("s") * 2 + lax.axis_index("c")
        base = wid * _RPW
        pltpu.sync_copy(scores_hbm, sc_v)
        lane = lax.iota(jnp.int32, _LANES)
        score = jnp.zeros((_LANES,), jnp.int32)
        for m in range(_N):
            score += plsc.load_gather(sc_v, [lane * _N + m])
        key = score * _N + (_N - 1 - lane)
        key_v[...] = key
        pos = jnp.zeros((_LANES,), jnp.int32)
        for m in range(_N):
            km = plsc.load_gather(key_v, [jnp.full((_LANES,), m, jnp.int32)])
            pos += (km > key).astype(jnp.int32)
        plsc.store_scatter(sel_v, [pos], lane)
        for i in range(_RPW // _LANES):
            t = base + i * _LANES + lane
            c = jnp.bitwise_and(t, _CLEAN - 1)
            j = lax.shift_right_logical(t, 3)
            src = j * _N + plsc.load_gather(sel_v, [c])
            off = i * _LANES
            idx_v[off // _CH, pl.ds(off % _CH, _LANES)] = src
        copies = [
            pltpu.async_copy(x_hbm.at[idx_v.at[ci]],
                             rows_v.at[pl.ds(ci * _CH, _CH)], sem)
            for ci in range(_NCH)
        ]
        for cp in copies:
            cp.wait()
        pltpu.sync_copy(rows_v, out_hbm.at[pl.ds(base, _RPW)])

    return select_gather


@functools.cache
def _select_gather_fn():
    return _make_select_gather()


def kernel(x):
    scores = _scores(x)
    return _select_gather_fn()(x, scores)
